# Initial kernel scaffold; baseline (speedup 1.0000x reference)
import jax
import jax.numpy as jnp
from jax.experimental import pallas as pl


def kernel(x, Wr, br, W1, b1, W2, b2):
    raise NotImplementedError("write your pallas kernel here")


# speedup vs baseline: 1.2576x; 1.2576x over previous
import jax
import jax.numpy as jnp
from jax.experimental import pallas as pl
from jax.experimental.pallas import tpu as pltpu

_B, _T, _D, _E, _K = 1, 2048, 1024, 8, 2
_H = 2 * _D
_N = _B * _T
_COEF = 0.0001
_HB = 512
_NJ = _H // _HB


def _router_body(x_ref, wr_ref, br_ref, we_ref, loss_ref):
    x = x_ref[...]
    logits = jnp.dot(x, wr_ref[...], preferred_element_type=jnp.float32)
    logits = logits + br_ref[...]
    m = jnp.max(logits, axis=1, keepdims=True)
    ex = jnp.exp(logits - m)
    g = ex / jnp.sum(ex, axis=1, keepdims=True)
    lane = jax.lax.broadcasted_iota(jnp.int32, (_N, _E), 1)
    m1 = jnp.max(g, axis=1, keepdims=True)
    i1 = jnp.min(jnp.where(g == m1, lane, _E), axis=1, keepdims=True)
    g2 = jnp.where(lane == i1, -1.0, g)
    m2 = jnp.max(g2, axis=1, keepdims=True)
    i2 = jnp.min(jnp.where(g2 == m2, lane, _E), axis=1, keepdims=True)
    we_ref[...] = (jnp.where(lane == i1, m1, 0.0)
                   + jnp.where(lane == i2, m2, 0.0))
    es = jnp.mean(g, axis=0, keepdims=True)
    diff = (1.0 / _E) - es
    loss_ref[0, 0] = jnp.mean(diff * diff) * _COEF


def _ffn_body(we_ref, x_ref, w1_ref, b1_ref, w2_ref, b2_ref, out_ref,
              yacc_ref):
    e = pl.program_id(0)
    j = pl.program_id(1)
    h = jnp.dot(x_ref[...], w1_ref[0], preferred_element_type=jnp.float32)
    h = h + b1_ref[0]
    h = jnp.where(h >= 0, h, 0.01 * h)
    contrib = jnp.dot(h, w2_ref[0], preferred_element_type=jnp.float32)

    @pl.when(j == 0)
    def _():
        yacc_ref[...] = contrib

    @pl.when(j != 0)
    def _():
        yacc_ref[...] = yacc_ref[...] + contrib

    @pl.when(j == _NJ - 1)
    def _():
        y = yacc_ref[...] + b2_ref[0]
        y = jnp.where(y >= 0, y, 0.01 * y)
        lane = jax.lax.broadcasted_iota(jnp.int32, (_N, _E), 1)
        wcol = jnp.sum(jnp.where(lane == e, we_ref[...], 0.0), axis=1,
                       keepdims=True)
        delta = wcol * y

        @pl.when(e == 0)
        def _():
            out_ref[...] = delta

        @pl.when(e != 0)
        def _():
            out_ref[...] = out_ref[...] + delta


def kernel(x, Wr, br, W1, b1, W2, b2):
    x2d = x.reshape(_N, _D)
    we, loss = pl.pallas_call(
        _router_body,
        out_shape=(
            jax.ShapeDtypeStruct((_N, _E), jnp.float32),
            jax.ShapeDtypeStruct((1, 1), jnp.float32),
        ),
        out_specs=(
            pl.BlockSpec((_N, _E), lambda: (0, 0)),
            pl.BlockSpec(memory_space=pltpu.SMEM),
        ),
    )(x2d, Wr, br.reshape(1, _E))

    out2d = pl.pallas_call(
        _ffn_body,
        grid=(_E, _NJ),
        in_specs=[
            pl.BlockSpec((_N, _E), lambda e, j: (0, 0)),
            pl.BlockSpec((_N, _D), lambda e, j: (0, 0)),
            pl.BlockSpec((1, _D, _HB), lambda e, j: (e, 0, j)),
            pl.BlockSpec((1, 1, _HB), lambda e, j: (e, 0, j)),
            pl.BlockSpec((1, _HB, _D), lambda e, j: (e, j, 0)),
            pl.BlockSpec((1, 1, _D), lambda e, j: (e, 0, 0)),
        ],
        out_specs=pl.BlockSpec((_N, _D), lambda e, j: (0, 0)),
        out_shape=jax.ShapeDtypeStruct((_N, _D), jnp.float32),
        scratch_shapes=[pltpu.VMEM((_N, _D), jnp.float32)],
        compiler_params=pltpu.CompilerParams(
            dimension_semantics=("arbitrary", "arbitrary")),
    )(we, x2d, W1, b1.reshape(_E, 1, _H), W2, b2.reshape(_E, 1, _D))

    return (out2d.reshape(_B, _T, _D), loss[0, 0])
